# baseline (speedup 1.0000x reference)
import functools

import jax
import jax.numpy as jnp
import numpy as np
from jax import lax
from jax.experimental import pallas as pl
from jax.experimental.pallas import tpu as pltpu
from jax.experimental.pallas import tpu_sc as plsc

_PATCH = 16
_NPS = 32

_KEEP_BITS_HEX = [
    0x36eadc9b, 0x6db41695, 0xab1ba7bb, 0x6ee7587b,
    0x16d82f89, 0x71d063b6, 0x69ab3a93, 0x7339a0b9,
    0x8e82277b, 0x14fdcc8a, 0x1e6a6284, 0xdf0e4208,
    0x243af85f, 0x1d7ccc04, 0xe52d395f, 0xc619ad56,
    0x2fd3344b, 0x450e09d3, 0x3bfa5e0d, 0x123fe3f5,
    0xf750ca43, 0xe8299b1c, 0x24baa733, 0x1d15fc6f,
    0x410732a4, 0xa48fd812, 0xe4ee24d4, 0xc6fbd063,
    0x33412a1d, 0x10e63c49, 0x7ed280a9, 0xf411ae0e,
]
_KEEP_BITS = np.array(_KEEP_BITS_HEX, dtype=np.uint32).view(np.int32)
_KEEP_PATCH = np.array(
    [[(b >> c) & 1 for c in range(_NPS)] for b in _KEEP_BITS_HEX],
    dtype=np.float32,
)
_KEEP_FULL = np.kron(_KEEP_PATCH, np.ones((_PATCH, _PATCH), np.float32))

_C, _H, _W = 96, 512, 512
_K_TC = 56
_C_SC = _C - _K_TC
_CC = 4
_NCH = _C_SC // _CC
_NRUN = _NCH


def _sc_fill(img_hbm, tbl_hbm, out_hbm, buf, tblv, gsem, ssem):
    wid = lax.axis_index("s") * 2 + lax.axis_index("c")
    r0 = wid * _PATCH

    pltpu.sync_copy(tbl_hbm, tblv)
    iota = lax.iota(jnp.int32, 16)
    lane = wid & 15
    half = lax.shift_right_logical(wid, 4)
    tv0 = tblv[pl.ds(0, 16)]
    tv1 = tblv[pl.ds(16, 16)]
    tv = jnp.where(jnp.full((16,), half, jnp.int32) == 0, tv0, tv1)
    mybits = jnp.where(iota == jnp.full((16,), lane, jnp.int32), tv, 0)
    keep_flags = [
        jnp.any((lax.shift_right_logical(mybits, jnp.int32(col)) & 1) == 1)
        for col in range(_NPS)
    ]

    def start_gathers(g, par):
        c0 = _K_TC + g * _CC
        b0 = par * _CC
        pltpu.async_copy(
            img_hbm.at[pl.ds(c0, _CC), pl.ds(r0, _PATCH), :],
            buf.at[pl.ds(b0, _CC)],
            gsem,
        )

    def drain_gathers():
        pltpu.make_async_copy(
            img_hbm.at[pl.ds(0, _CC), pl.ds(0, _PATCH), :],
            buf.at[pl.ds(0, _CC)],
            gsem,
        ).wait()

    zero16 = jnp.zeros((16,), jnp.float32)

    def erase(par):
        b0 = par * _CC
        for col in range(_NPS):
            if True:
                @pl.when(jnp.logical_not(keep_flags[col]))
                def _():
                    for cc in range(_CC):
                        for r in range(_PATCH):
                            buf[b0 + cc, r, pl.ds(col * _PATCH, _PATCH)] = (
                                zero16)

    def start_scatter(g, par):
        c0 = g * _CC
        b0 = par * _CC
        pltpu.async_copy(
            buf.at[pl.ds(b0, _CC)],
            out_hbm.at[pl.ds(c0, _CC), pl.ds(r0, _PATCH), :],
            ssem,
        )

    def wait_scatter():
        pltpu.make_async_copy(
            buf.at[pl.ds(0, _CC)],
            out_hbm.at[pl.ds(0, _CC), pl.ds(0, _PATCH), :],
            ssem,
        ).wait()

    start_gathers(jnp.int32(0), jnp.int32(0))

    def chunk(g, carry):
        par = g & 1
        @pl.when(g >= 1)
        def _():
            wait_scatter()
        @pl.when(g + 1 < _NRUN)
        def _():
            start_gathers(g + 1, 1 - par)
        drain_gathers()
        erase(par)
        start_scatter(g, par)
        return carry

    lax.fori_loop(0, _NRUN, chunk, jnp.int32(0))
    wait_scatter()


def _tc_body(mask_ref, img_ref, out_ref):
    out_ref[...] = img_ref[...] * mask_ref[...][None, :, :]


def kernel(img):
    c, h, w = img.shape
    cb = 8
    mask = jnp.asarray(_KEEP_FULL)
    tc_out = pl.pallas_call(
        _tc_body,
        grid=(_K_TC // cb,),
        in_specs=[
            pl.BlockSpec((h, w), lambda i: (0, 0)),
            pl.BlockSpec((cb, h, w), lambda i: (i, 0, 0)),
        ],
        out_specs=pl.BlockSpec((cb, h, w), lambda i: (i, 0, 0)),
        out_shape=jax.ShapeDtypeStruct((_K_TC, h, w), img.dtype),
    )(mask, img)
    tbl = jnp.asarray(_KEEP_BITS)
    fn = pl.kernel(
        _sc_fill,
        out_type=jax.ShapeDtypeStruct((_C_SC, h, w), img.dtype),
        mesh=plsc.VectorSubcoreMesh(core_axis_name="c", subcore_axis_name="s"),
        compiler_params=pltpu.CompilerParams(
            use_tc_tiling_on_sc=True, needs_layout_passes=False),
        scratch_types=[
            pltpu.VMEM((2 * _CC, _PATCH, w), img.dtype),
            pltpu.VMEM((_NPS,), jnp.int32),
            pltpu.SemaphoreType.DMA,
            pltpu.SemaphoreType.DMA,
        ],
    )
    sc_out = fn(img, tbl)
    return jnp.concatenate([tc_out, sc_out], axis=0)


# speedup vs baseline: 2.3801x; 2.3801x over previous
import jax
import jax.numpy as jnp
import numpy as np
from jax.experimental import pallas as pl

_PATCH = 16
_NPS = 32

_KEEP_BITS_HEX = [
    0x36eadc9b, 0x6db41695, 0xab1ba7bb, 0x6ee7587b,
    0x16d82f89, 0x71d063b6, 0x69ab3a93, 0x7339a0b9,
    0x8e82277b, 0x14fdcc8a, 0x1e6a6284, 0xdf0e4208,
    0x243af85f, 0x1d7ccc04, 0xe52d395f, 0xc619ad56,
    0x2fd3344b, 0x450e09d3, 0x3bfa5e0d, 0x123fe3f5,
    0xf750ca43, 0xe8299b1c, 0x24baa733, 0x1d15fc6f,
    0x410732a4, 0xa48fd812, 0xe4ee24d4, 0xc6fbd063,
    0x33412a1d, 0x10e63c49, 0x7ed280a9, 0xf411ae0e,
]

_KEEP_PATCH = np.array(
    [[(b >> c) & 1 for c in range(_NPS)] for b in _KEEP_BITS_HEX],
    dtype=np.float32,
)
_KEEP_FULL = np.kron(_KEEP_PATCH, np.ones((_PATCH, _PATCH), np.float32))


def _body(mask_ref, img_ref, out_ref):
    out_ref[...] = img_ref[...] * mask_ref[...][None, :, :]


def kernel(img):
    c, h, w = img.shape
    cb = 4
    mask = jnp.asarray(_KEEP_FULL)
    return pl.pallas_call(
        _body,
        grid=(c // cb,),
        in_specs=[
            pl.BlockSpec((h, w), lambda i: (0, 0)),
            pl.BlockSpec((cb, h, w), lambda i: (i, 0, 0)),
        ],
        out_specs=pl.BlockSpec((cb, h, w), lambda i: (i, 0, 0)),
        out_shape=jax.ShapeDtypeStruct((c, h, w), img.dtype),
    )(mask, img)


# speedup vs baseline: 2.4603x; 1.0337x over previous
import jax
import jax.numpy as jnp
import numpy as np
from jax.experimental import pallas as pl

_PATCH = 16
_NPS = 32

_KEEP_BITS_HEX = [
    0x36eadc9b, 0x6db41695, 0xab1ba7bb, 0x6ee7587b,
    0x16d82f89, 0x71d063b6, 0x69ab3a93, 0x7339a0b9,
    0x8e82277b, 0x14fdcc8a, 0x1e6a6284, 0xdf0e4208,
    0x243af85f, 0x1d7ccc04, 0xe52d395f, 0xc619ad56,
    0x2fd3344b, 0x450e09d3, 0x3bfa5e0d, 0x123fe3f5,
    0xf750ca43, 0xe8299b1c, 0x24baa733, 0x1d15fc6f,
    0x410732a4, 0xa48fd812, 0xe4ee24d4, 0xc6fbd063,
    0x33412a1d, 0x10e63c49, 0x7ed280a9, 0xf411ae0e,
]

_KEEP_PATCH = np.array(
    [[(b >> c) & 1 for c in range(_NPS)] for b in _KEEP_BITS_HEX],
    dtype=np.float32,
)
_KEEP_FULL = np.kron(_KEEP_PATCH, np.ones((_PATCH, _PATCH), np.float32))


def _body(mask_ref, img_ref, out_ref):
    out_ref[...] = img_ref[...] * mask_ref[...][None, :, :]


def kernel(img):
    c, h, w = img.shape
    cb = 12
    mask = jnp.asarray(_KEEP_FULL)
    return pl.pallas_call(
        _body,
        grid=(c // cb,),
        in_specs=[
            pl.BlockSpec((h, w), lambda i: (0, 0)),
            pl.BlockSpec((cb, h, w), lambda i: (i, 0, 0)),
        ],
        out_specs=pl.BlockSpec((cb, h, w), lambda i: (i, 0, 0)),
        out_shape=jax.ShapeDtypeStruct((c, h, w), img.dtype),
    )(mask, img)
